# baseline (speedup 1.0000x reference)
import jax
import jax.numpy as jnp
from jax import lax
from jax.experimental import pallas as pl
from jax.experimental.pallas import tpu as pltpu
from jax.experimental.pallas import tpu_sc as plsc

N_NODES = 10000
N_EDGES = 320000
D = 128

NC = 2
NS = 16
NW = NC * NS
CHUNK = 128
EDGES_PER_W = N_EDGES // NW
NCHUNK = (EDGES_PER_W + CHUNK - 1) // CHUNK
EDGES_PAD = NCHUNK * CHUNK
ACC_ROWS = 10112
ZROWS = ACC_ROWS // NS


def _mm_body(h_ref, w_ref, o_ref):
    o_ref[...] = jnp.dot(h_ref[...], w_ref[...],
                         preferred_element_type=jnp.float32)


def _matmul(h, W):
    return pl.pallas_call(
        _mm_body,
        grid=(10,),
        in_specs=[
            pl.BlockSpec((N_NODES // 10, D), lambda i: (i, 0)),
            pl.BlockSpec((D, D), lambda i: (0, 0)),
        ],
        out_specs=pl.BlockSpec((N_NODES // 10, D), lambda i: (i, 0)),
        out_shape=jax.ShapeDtypeStruct((N_NODES, D), jnp.float32),
    )(h, W)


def _scatter_body(h2_hbm, src_hbm, dst_hbm, zeros_hbm, parts_hbm,
                  acc, src_v, dst_v, rows_v, sem):
    c = lax.axis_index("c")
    s = lax.axis_index("s")
    wid = c * NS + s

    pltpu.sync_copy(zeros_hbm, acc.at[pl.ds(s * ZROWS, ZROWS)])

    pltpu.sync_copy(src_hbm.at[wid], src_v)
    pltpu.sync_copy(dst_hbm.at[wid], dst_v)

    plsc.subcore_barrier()

    for j in range(NCHUNK):
        pltpu.async_copy(h2_hbm.at[src_v.at[j]], rows_v, sem).wait()
        pltpu.sync_copy(rows_v, acc.at[dst_v.at[j]], add=True)

    plsc.subcore_barrier()

    pltpu.sync_copy(acc.at[pl.ds(s * ZROWS, ZROWS)],
                    parts_hbm.at[c, pl.ds(s * ZROWS, ZROWS)])


def _message_pass(h2, src_p, dst_p, zeros):
    mesh = plsc.VectorSubcoreMesh(core_axis_name="c", subcore_axis_name="s")
    k = pl.kernel(
        _scatter_body,
        out_type=jax.ShapeDtypeStruct((NC, ACC_ROWS, D), jnp.float32),
        mesh=mesh,
        scratch_types=[
            pltpu.VMEM_SHARED((ACC_ROWS, D), jnp.float32),
            pltpu.VMEM((NCHUNK, CHUNK), jnp.int32),
            pltpu.VMEM((NCHUNK, CHUNK), jnp.int32),
            pltpu.VMEM((CHUNK, D), jnp.float32),
            pltpu.SemaphoreType.DMA,
        ],
    )
    return k(h2, src_p, dst_p, zeros)


def _fin_body(p_ref, b_ref, o_ref):
    o_ref[...] = jnp.maximum(p_ref[0] + p_ref[1] + b_ref[...], 0.0)


def _finalize(parts, b2):
    return pl.pallas_call(
        _fin_body,
        grid=(10,),
        in_specs=[
            pl.BlockSpec((NC, N_NODES // 10, D), lambda i: (0, i, 0)),
            pl.BlockSpec((1, D), lambda i: (0, 0)),
        ],
        out_specs=pl.BlockSpec((N_NODES // 10, D), lambda i: (i, 0)),
        out_shape=jax.ShapeDtypeStruct((N_NODES, D), jnp.float32),
    )(parts, b2)


@jax.jit
def kernel(edge_index, h, W, b):
    src = edge_index[0].reshape(NW, EDGES_PER_W)
    dst = edge_index[1].reshape(NW, EDGES_PER_W)
    pad = EDGES_PAD - EDGES_PER_W
    src_p = jnp.pad(src, ((0, 0), (0, pad))).reshape(NW, NCHUNK, CHUNK)
    dst_p = jnp.pad(dst, ((0, 0), (0, pad)),
                    constant_values=N_NODES).reshape(NW, NCHUNK, CHUNK)
    zeros = jnp.zeros((ZROWS, D), jnp.float32)

    h2 = _matmul(h, W)
    parts = _message_pass(h2, src_p, dst_p, zeros)
    return _finalize(parts, b.reshape(1, D))


# speedup vs baseline: 1.0198x; 1.0198x over previous
import jax
import jax.numpy as jnp
from jax import lax
from jax.experimental import pallas as pl
from jax.experimental.pallas import tpu as pltpu
from jax.experimental.pallas import tpu_sc as plsc

N_NODES = 10000
N_EDGES = 320000
D = 128

NC = 2
NS = 16
NW = NC * NS
CHUNK = 128
EDGES_PER_W = N_EDGES // NW
NCHUNK = (EDGES_PER_W + CHUNK - 1) // CHUNK
EDGES_PAD = NCHUNK * CHUNK
ACC_ROWS = 10112
ZROWS = ACC_ROWS // NS


def _mm_body(h_ref, w_ref, o_ref):
    o_ref[...] = jnp.dot(h_ref[...], w_ref[...],
                         preferred_element_type=jnp.float32)


def _matmul(h, W):
    return pl.pallas_call(
        _mm_body,
        out_shape=jax.ShapeDtypeStruct((N_NODES, D), jnp.float32),
    )(h, W)


def _scatter_body(h2_hbm, src_hbm, dst_hbm, zeros_hbm, parts_hbm,
                  acc, src_v, dst_v, rows_v, sem):
    c = lax.axis_index("c")
    s = lax.axis_index("s")
    wid = c * NS + s

    pltpu.sync_copy(zeros_hbm, acc.at[pl.ds(s * ZROWS, ZROWS)])

    pltpu.sync_copy(src_hbm.at[wid], src_v)
    pltpu.sync_copy(dst_hbm.at[wid], dst_v)

    plsc.subcore_barrier()

    for j in range(NCHUNK):
        pltpu.async_copy(h2_hbm.at[src_v.at[j]], rows_v, sem).wait()
        pltpu.sync_copy(rows_v, acc.at[dst_v.at[j]], add=True)

    plsc.subcore_barrier()

    pltpu.sync_copy(acc.at[pl.ds(s * ZROWS, ZROWS)],
                    parts_hbm.at[c, pl.ds(s * ZROWS, ZROWS)])


def _message_pass(h2, src_p, dst_p, zeros):
    mesh = plsc.VectorSubcoreMesh(core_axis_name="c", subcore_axis_name="s")
    k = pl.kernel(
        _scatter_body,
        out_type=jax.ShapeDtypeStruct((NC, ACC_ROWS, D), jnp.float32),
        mesh=mesh,
        scratch_types=[
            pltpu.VMEM_SHARED((ACC_ROWS, D), jnp.float32),
            pltpu.VMEM((NCHUNK, CHUNK), jnp.int32),
            pltpu.VMEM((NCHUNK, CHUNK), jnp.int32),
            pltpu.VMEM((CHUNK, D), jnp.float32),
            pltpu.SemaphoreType.DMA,
        ],
    )
    return k(h2, src_p, dst_p, zeros)


def _fin_body(p_ref, b_ref, o_ref):
    o_ref[...] = jnp.maximum(p_ref[0] + p_ref[1] + b_ref[...], 0.0)


def _finalize(parts, b2):
    return pl.pallas_call(
        _fin_body,
        grid=(1,),
        in_specs=[
            pl.BlockSpec((NC, N_NODES, D), lambda i: (0, 0, 0)),
            pl.BlockSpec((1, D), lambda i: (0, 0)),
        ],
        out_specs=pl.BlockSpec((N_NODES, D), lambda i: (0, 0)),
        out_shape=jax.ShapeDtypeStruct((N_NODES, D), jnp.float32),
    )(parts, b2)


@jax.jit
def kernel(edge_index, h, W, b):
    src = edge_index[0].reshape(NW, EDGES_PER_W)
    dst = edge_index[1].reshape(NW, EDGES_PER_W)
    pad = EDGES_PAD - EDGES_PER_W
    src_p = jnp.pad(src, ((0, 0), (0, pad))).reshape(NW, NCHUNK, CHUNK)
    dst_p = jnp.pad(dst, ((0, 0), (0, pad)),
                    constant_values=N_NODES).reshape(NW, NCHUNK, CHUNK)
    zeros = jnp.zeros((ZROWS, D), jnp.float32)

    h2 = _matmul(h, W)
    parts = _message_pass(h2, src_p, dst_p, zeros)
    return _finalize(parts, b.reshape(1, D))


# speedup vs baseline: 1.0241x; 1.0043x over previous
import jax
import jax.numpy as jnp
from jax import lax
from jax.experimental import pallas as pl
from jax.experimental.pallas import tpu as pltpu
from jax.experimental.pallas import tpu_sc as plsc

N_NODES = 10000
N_EDGES = 320000
D = 128

NC = 2
NS = 16
NW = NC * NS
CHUNK = 128
EDGES_PER_W = N_EDGES // NW
NCHUNK = (EDGES_PER_W + CHUNK - 1) // CHUNK
EDGES_PAD = NCHUNK * CHUNK
ACC_ROWS = 10112
ZROWS = ACC_ROWS // NS


def _mm_body(h_ref, w_ref, o_ref):
    o_ref[...] = jnp.dot(h_ref[...], w_ref[...],
                         preferred_element_type=jnp.float32)


def _matmul(h, W):
    return pl.pallas_call(
        _mm_body,
        out_shape=jax.ShapeDtypeStruct((N_NODES, D), jnp.float32),
    )(h, W)


def _scatter_body(h2_hbm, src_hbm, dst_hbm, zeros_hbm, parts_hbm,
                  acc, src_v, dst_v, rows_v, sem):
    c = lax.axis_index("c")
    s = lax.axis_index("s")
    wid = c * NS + s

    pltpu.async_copy(zeros_hbm, acc.at[pl.ds(s * ZROWS, ZROWS)], sem)
    pltpu.async_copy(src_hbm.at[wid], src_v, sem)
    pltpu.async_copy(dst_hbm.at[wid], dst_v, sem)
    pltpu.make_async_copy(zeros_hbm, acc.at[pl.ds(s * ZROWS, ZROWS)],
                          sem).wait()
    pltpu.make_async_copy(src_hbm.at[wid], src_v, sem).wait()
    pltpu.make_async_copy(dst_hbm.at[wid], dst_v, sem).wait()

    plsc.subcore_barrier()

    for j in range(NCHUNK):
        pltpu.async_copy(h2_hbm.at[src_v.at[j]], rows_v, sem).wait()
        pltpu.sync_copy(rows_v, acc.at[dst_v.at[j]], add=True)

    plsc.subcore_barrier()

    pltpu.sync_copy(acc.at[pl.ds(s * ZROWS, ZROWS)],
                    parts_hbm.at[c, pl.ds(s * ZROWS, ZROWS)])


def _message_pass(h2, src_p, dst_p, zeros):
    mesh = plsc.VectorSubcoreMesh(core_axis_name="c", subcore_axis_name="s")
    k = pl.kernel(
        _scatter_body,
        out_type=jax.ShapeDtypeStruct((NC, ACC_ROWS, D), jnp.float32),
        mesh=mesh,
        scratch_types=[
            pltpu.VMEM_SHARED((ACC_ROWS, D), jnp.float32),
            pltpu.VMEM((NCHUNK, CHUNK), jnp.int32),
            pltpu.VMEM((NCHUNK, CHUNK), jnp.int32),
            pltpu.VMEM((CHUNK, D), jnp.float32),
            pltpu.SemaphoreType.DMA,
        ],
    )
    return k(h2, src_p, dst_p, zeros)


def _fin_body(p_ref, b_ref, o_ref):
    o_ref[...] = jnp.maximum(p_ref[0] + p_ref[1] + b_ref[...], 0.0)


def _finalize(parts, b2):
    return pl.pallas_call(
        _fin_body,
        grid=(1,),
        in_specs=[
            pl.BlockSpec((NC, N_NODES, D), lambda i: (0, 0, 0)),
            pl.BlockSpec((1, D), lambda i: (0, 0)),
        ],
        out_specs=pl.BlockSpec((N_NODES, D), lambda i: (0, 0)),
        out_shape=jax.ShapeDtypeStruct((N_NODES, D), jnp.float32),
    )(parts, b2)


@jax.jit
def kernel(edge_index, h, W, b):
    src = edge_index[0].reshape(NW, EDGES_PER_W)
    dst = edge_index[1].reshape(NW, EDGES_PER_W)
    pad = EDGES_PAD - EDGES_PER_W
    src_p = jnp.pad(src, ((0, 0), (0, pad))).reshape(NW, NCHUNK, CHUNK)
    dst_p = jnp.pad(dst, ((0, 0), (0, pad)),
                    constant_values=N_NODES).reshape(NW, NCHUNK, CHUNK)
    zeros = jnp.zeros((ZROWS, D), jnp.float32)

    h2 = _matmul(h, W)
    parts = _message_pass(h2, src_p, dst_p, zeros)
    return _finalize(parts, b.reshape(1, D))


# speedup vs baseline: 1.0344x; 1.0100x over previous
import jax
import jax.numpy as jnp
from jax import lax
from jax.experimental import pallas as pl
from jax.experimental.pallas import tpu as pltpu
from jax.experimental.pallas import tpu_sc as plsc

N_NODES = 10000
N_EDGES = 320000
D = 128

NC = 2
NS = 16
NW = NC * NS
CHUNK = 128
EDGES_PER_W = N_EDGES // NW
NCHUNK = (EDGES_PER_W + CHUNK - 1) // CHUNK
EDGES_PAD = NCHUNK * CHUNK
ACC_ROWS = 10112
ZROWS = ACC_ROWS // NS


def _mm_body(h_ref, w_ref, o_ref):
    o_ref[...] = jnp.dot(h_ref[...], w_ref[...],
                         preferred_element_type=jnp.float32)


def _matmul(h, W):
    return pl.pallas_call(
        _mm_body,
        out_shape=jax.ShapeDtypeStruct((N_NODES, D), jnp.float32),
    )(h, W)


def _scatter_body(h2_hbm, src_hbm, dst_hbm, zeros_hbm, parts_hbm,
                  acc, src_v, dst_v, rows_v, sem):
    c = lax.axis_index("c")
    s = lax.axis_index("s")
    wid = c * NS + s

    pltpu.async_copy(zeros_hbm, acc.at[pl.ds(s * ZROWS, ZROWS)], sem)
    pltpu.async_copy(src_hbm.at[wid], src_v, sem)
    pltpu.async_copy(dst_hbm.at[wid], dst_v, sem)
    pltpu.make_async_copy(zeros_hbm, acc.at[pl.ds(s * ZROWS, ZROWS)],
                          sem).wait()
    pltpu.make_async_copy(src_hbm.at[wid], src_v, sem).wait()
    pltpu.make_async_copy(dst_hbm.at[wid], dst_v, sem).wait()

    plsc.subcore_barrier()

    def step(j, carry):
        pltpu.async_copy(h2_hbm.at[src_v.at[j]], rows_v, sem).wait()
        pltpu.sync_copy(rows_v, acc.at[dst_v.at[j]], add=True)
        return carry

    lax.fori_loop(0, NCHUNK, step, 0)

    plsc.subcore_barrier()

    pltpu.sync_copy(acc.at[pl.ds(s * ZROWS, ZROWS)],
                    parts_hbm.at[c, pl.ds(s * ZROWS, ZROWS)])


def _message_pass(h2, src_p, dst_p, zeros):
    mesh = plsc.VectorSubcoreMesh(core_axis_name="c", subcore_axis_name="s")
    k = pl.kernel(
        _scatter_body,
        out_type=jax.ShapeDtypeStruct((NC, ACC_ROWS, D), jnp.float32),
        mesh=mesh,
        scratch_types=[
            pltpu.VMEM_SHARED((ACC_ROWS, D), jnp.float32),
            pltpu.VMEM((NCHUNK, CHUNK), jnp.int32),
            pltpu.VMEM((NCHUNK, CHUNK), jnp.int32),
            pltpu.VMEM((CHUNK, D), jnp.float32),
            pltpu.SemaphoreType.DMA,
        ],
    )
    return k(h2, src_p, dst_p, zeros)


def _fin_body(p_ref, b_ref, o_ref):
    o_ref[...] = jnp.maximum(p_ref[0] + p_ref[1] + b_ref[...], 0.0)


def _finalize(parts, b2):
    return pl.pallas_call(
        _fin_body,
        grid=(1,),
        in_specs=[
            pl.BlockSpec((NC, N_NODES, D), lambda i: (0, 0, 0)),
            pl.BlockSpec((1, D), lambda i: (0, 0)),
        ],
        out_specs=pl.BlockSpec((N_NODES, D), lambda i: (0, 0)),
        out_shape=jax.ShapeDtypeStruct((N_NODES, D), jnp.float32),
    )(parts, b2)


@jax.jit
def kernel(edge_index, h, W, b):
    src = edge_index[0].reshape(NW, EDGES_PER_W)
    dst = edge_index[1].reshape(NW, EDGES_PER_W)
    pad = EDGES_PAD - EDGES_PER_W
    src_p = jnp.pad(src, ((0, 0), (0, pad))).reshape(NW, NCHUNK, CHUNK)
    dst_p = jnp.pad(dst, ((0, 0), (0, pad)),
                    constant_values=N_NODES).reshape(NW, NCHUNK, CHUNK)
    zeros = jnp.zeros((ZROWS, D), jnp.float32)

    h2 = _matmul(h, W)
    parts = _message_pass(h2, src_p, dst_p, zeros)
    return _finalize(parts, b.reshape(1, D))
